# baseline (speedup 1.0000x reference)
import jax
import jax.numpy as jnp
from jax import lax
from jax.experimental import pallas as pl
from jax.experimental.pallas import tpu as pltpu
from jax.experimental.pallas import tpu_sc as plsc

VOCAB = 1_000_000
HIDDEN = 32
BATCH = 4096
HIST = 50
NC, NS, LANES = 2, 16, 16
NW = NC * NS
BPW = BATCH // NW
NGRP = BPW // LANES
TAIL = (VOCAB // 128) * 128
NBIG = VOCAB // 4
SLABW = 512
NSLAB = TAIL // SLABW
SPAD = SLABW + 1


def _wid():
    return lax.axis_index("s") * NC + lax.axis_index("c")


def _mesh():
    return plsc.VectorSubcoreMesh(core_axis_name="c", subcore_axis_name="s",
                                  num_cores=NC, num_subcores=NS)


def _ctx_body(ctxT_hbm, tbl4_hbm, con_hbm,
              ctx_v, big_v, rows0, rows1, acc, sem0, sem1):
    base = _wid() * BPW
    iota = lax.iota(jnp.int32, LANES)
    pltpu.sync_copy(ctxT_hbm.at[:, pl.ds(base, BPW)], ctx_v)

    @pl.loop(0, HIST)
    def _prep(h):
        for c in range(NGRP):
            sl = pl.ds(c * LANES, LANES)
            big_v[h, sl] = lax.shift_right_logical(ctx_v[h, sl], 2)

    zeros = jnp.zeros((LANES,), jnp.float32)

    @pl.loop(0, BPW)
    def _zero(i):
        acc[i, pl.ds(0, LANES)] = zeros
        acc[i, pl.ds(LANES, LANES)] = zeros

    def accum(h, rows):
        for c in range(NGRP):
            chunk = ctx_v[h, pl.ds(c * LANES, LANES)]
            for l in range(LANES):
                slot = c * LANES + l
                off = lax.shift_left(lax.bitwise_and(chunk[l], 3), 5)
                ids = jnp.full((LANES,), slot, jnp.int32)
                offs = iota + off
                lo = plsc.load_gather(rows, [ids, offs])
                hi = plsc.load_gather(rows, [ids, offs + LANES])
                plsc.addupdate(acc.at[slot, pl.ds(0, LANES)], lo)
                plsc.addupdate(acc.at[slot, pl.ds(LANES, LANES)], hi)

    pltpu.async_copy(tbl4_hbm.at[big_v.at[0]], rows0, sem0)
    pltpu.async_copy(tbl4_hbm.at[big_v.at[1]], rows1, sem1)

    @pl.loop(0, HIST, step=2)
    def _h(h):
        for b, (rows, sem) in enumerate(((rows0, sem0), (rows1, sem1))):
            hc = h + b
            pltpu.make_async_copy(tbl4_hbm.at[big_v.at[hc]], rows, sem).wait()
            accum(hc, rows)

            @pl.when(hc + 2 < HIST)
            def _next():
                pltpu.async_copy(tbl4_hbm.at[big_v.at[hc + 2]], rows, sem)

    pltpu.sync_copy(acc, con_hbm.at[pl.ds(base, BPW)])


def _ctx_gather(ctxT, tbl4):
    f = pl.kernel(
        _ctx_body,
        out_type=jax.ShapeDtypeStruct((BATCH, HIDDEN), jnp.float32),
        mesh=_mesh(),
        compiler_params=pltpu.CompilerParams(needs_layout_passes=False),
        scratch_types=[
            pltpu.VMEM((HIST, BPW), jnp.int32),
            pltpu.VMEM((HIST, BPW), jnp.int32),
            pltpu.VMEM((BPW, 128), jnp.float32),
            pltpu.VMEM((BPW, 128), jnp.float32),
            pltpu.VMEM((BPW, HIDDEN), jnp.float32),
            pltpu.SemaphoreType.DMA,
            pltpu.SemaphoreType.DMA,
        ],
    )
    return f(ctxT, tbl4)


def _tgt_body(t_hbm, outT_hbm, tgt_hbm,
              tidx_v, blk0, blk1, tgt_v, sem0, sem1):
    base = _wid() * BPW
    pltpu.sync_copy(t_hbm.at[pl.ds(base, BPW)], tidx_v)
    iota = lax.iota(jnp.int32, LANES)

    def block_base(vs):
        vc = lax.min(vs, TAIL - 1)
        return pl.multiple_of(
            lax.shift_left(lax.shift_right_logical(vc, 7), 7), 128)

    def fetch(vs, blk, sem):
        pltpu.async_copy(outT_hbm.at[:, pl.ds(block_base(vs), 128)], blk, sem)

    def wait(blk, sem):
        pltpu.make_async_copy(outT_hbm.at[:, pl.ds(0, 128)], blk, sem).wait()

    def select(i, vs, blk):
        off = lax.min(vs, TAIL - 1) - block_base(vs)
        lo = plsc.load_gather(blk, [iota, jnp.full((LANES,), off, jnp.int32)])
        hi = plsc.load_gather(blk, [iota + LANES,
                                    jnp.full((LANES,), off, jnp.int32)])
        tgt_v[i, pl.ds(0, LANES)] = lo
        tgt_v[i, pl.ds(LANES, LANES)] = hi

    @pl.loop(0, NGRP)
    def _g(c):
        chunk = tidx_v[pl.ds(c * LANES, LANES)]
        fetch(chunk[0], blk0, sem0)
        for l in range(LANES):
            if l + 1 < LANES:
                fetch(chunk[l + 1], (blk0, blk1)[(l + 1) % 2],
                      (sem0, sem1)[(l + 1) % 2])
            blk, sem = (blk0, blk1)[l % 2], (sem0, sem1)[l % 2]
            wait(blk, sem)
            select(c * LANES + l, chunk[l], blk)

    pltpu.sync_copy(tgt_v, tgt_hbm.at[pl.ds(base, BPW)])


def _tgt_gather(t_flat, out_embT):
    f = pl.kernel(
        _tgt_body,
        out_type=jax.ShapeDtypeStruct((BATCH, HIDDEN), jnp.float32),
        mesh=_mesh(),
        compiler_params=pltpu.CompilerParams(needs_layout_passes=False),
        scratch_types=[
            pltpu.VMEM((BPW,), jnp.int32),
            pltpu.VMEM((HIDDEN, 128), jnp.float32),
            pltpu.VMEM((HIDDEN, 128), jnp.float32),
            pltpu.VMEM((BPW, HIDDEN), jnp.float32),
            pltpu.SemaphoreType.DMA,
            pltpu.SemaphoreType.DMA,
        ],
    )
    return f(t_flat, out_embT)


def _mm_body(a_ref, b_ref, o_ref):
    o_ref[...] = lax.dot_general(a_ref[...], b_ref[...],
                                 (((1,), (1,)), ((), ())),
                                 preferred_element_type=jnp.float32)


def _tc_matmul(con, tgt):
    blk = 1024
    return pl.pallas_call(
        _mm_body,
        grid=(BATCH // blk, BATCH // blk),
        in_specs=[pl.BlockSpec((blk, HIDDEN), lambda i, j: (i, 0)),
                  pl.BlockSpec((blk, HIDDEN), lambda i, j: (j, 0))],
        out_specs=pl.BlockSpec((blk, blk), lambda i, j: (i, j)),
        out_shape=jax.ShapeDtypeStruct((BATCH, BATCH), jnp.float32),
    )(con, tgt)


def kernel(contexts, t, in_emb, out_emb):
    ctxT = contexts.T
    t_flat = t.reshape(BATCH)
    out_embT = out_emb.T
    tail_tab = out_emb[TAIL:]

    tbl4 = in_emb.reshape(NBIG, 128)
    con = _ctx_gather(ctxT, tbl4)
    tgt = _tgt_gather(t_flat, out_embT)
    tail_rows = jnp.take(tail_tab,
                         jnp.clip(t_flat - TAIL, 0, VOCAB - TAIL - 1), axis=0)
    tgt = jnp.where((t_flat >= TAIL)[:, None], tail_rows, tgt)
    return _tc_matmul(con, tgt)
